# Initial kernel scaffold; baseline (speedup 1.0000x reference)
import jax
import jax.numpy as jnp
from jax.experimental import pallas as pl


def kernel(inputs, mask, W_ih, W_hh, b_ih, b_hh):
    raise NotImplementedError("write your pallas kernel here")


# speedup vs baseline: 7.2056x; 7.2056x over previous
import functools

import jax
import jax.numpy as jnp
from jax import lax
from jax.experimental import pallas as pl
from jax.experimental.pallas import tpu as pltpu
from jax.experimental.pallas import tpu_sc as plsc

B, S, D, H = 16, 2048, 512, 512
G4 = 4 * H
CHUNK = 64


def _sc_sort(maskT_i32):
    mesh = plsc.VectorSubcoreMesh(core_axis_name="c", subcore_axis_name="s")

    @functools.partial(
        pl.kernel,
        mesh=mesh,
        out_type=(
            jax.ShapeDtypeStruct((B,), jnp.int32),
            jax.ShapeDtypeStruct((B,), jnp.int32),
        ),
        scratch_types=[
            pltpu.VMEM((128, B), jnp.int32),
            pltpu.VMEM((B,), jnp.int32),
            pltpu.VMEM((B,), jnp.int32),
        ],
    )
    def k(mask_hbm, rest_hbm, len_hbm, mask_v, rest_v, len_v):
        wid = lax.axis_index("s") * 2 + lax.axis_index("c")

        @pl.when(wid == 0)
        def _():
            lengths = jnp.zeros((B,), jnp.int32)
            for c in range(S // 128):
                pltpu.sync_copy(mask_hbm.at[pl.ds(c * 128, 128)], mask_v)

                def body(t, acc):
                    return acc + mask_v[t, :]

                lengths = lax.fori_loop(0, 128, body, lengths)
            idx = lax.iota(jnp.int32, B)
            keys = lengths * B + (B - 1) - idx
            rank = jnp.zeros((B,), jnp.int32)
            one = jnp.ones((B,), jnp.int32)
            zero = jnp.zeros((B,), jnp.int32)
            for j in range(B):
                kj = keys[j]
                rank = rank + jnp.where(kj > keys, one, zero)
            rest_v[...] = rank
            len_v[...] = lengths
            pltpu.sync_copy(rest_v, rest_hbm)
            pltpu.sync_copy(len_v, len_hbm)

    return k(maskT_i32)


def _lstm_body(x_ref, len_ref, rest_ref, wih_ref, whh_ref, bih_ref, bhh_ref,
               out_ref, hf_ref, cf_ref, h_s, c_s, a_s, res_s):
    i = pl.program_id(0)
    nt = pl.num_programs(0)

    @pl.when(i == 0)
    def _():
        h_s[...] = jnp.zeros((B, H), jnp.float32)
        c_s[...] = jnp.zeros((B, H), jnp.float32)

    xt = jnp.swapaxes(x_ref[...], 0, 1)
    x2 = xt.reshape(CHUNK * B, D)
    bias = bih_ref[...] + bhh_ref[...]
    a = lax.dot_general(x2, wih_ref[...], (((1,), (1,)), ((), ())),
                        preferred_element_type=jnp.float32) + bias
    a_s[...] = a.reshape(CHUNK, B, G4)

    lens = len_ref[...]
    whh = whh_ref[...]
    t0 = (i * CHUNK).astype(jnp.float32)

    def step(t, carry):
        h, c = carry
        gates = a_s[t] + lax.dot_general(
            h, whh, (((1,), (1,)), ((), ())), preferred_element_type=jnp.float32)
        ig = jax.nn.sigmoid(gates[:, 0:H])
        fg = jax.nn.sigmoid(gates[:, H:2 * H])
        gg = jnp.tanh(gates[:, 2 * H:3 * H])
        og = jax.nn.sigmoid(gates[:, 3 * H:4 * H])
        c_new = fg * c + ig * gg
        h_new = og * jnp.tanh(c_new)
        valid = lens > (t0 + t.astype(jnp.float32))
        res_s[:, t, :] = jnp.where(valid, h_new, 0.0)
        return (jnp.where(valid, h_new, h), jnp.where(valid, c_new, c))

    h_fin, c_fin = lax.fori_loop(0, CHUNK, step, (h_s[...], c_s[...]))
    h_s[...] = h_fin
    c_s[...] = c_fin

    rows = lax.broadcasted_iota(jnp.int32, (B, B), 0)
    perm = (rows == rest_ref[...]).astype(jnp.float32)
    res = res_s[...].reshape(B, CHUNK * H)
    out_ref[...] = jnp.dot(perm, res,
                           preferred_element_type=jnp.float32).reshape(B, CHUNK, H)

    @pl.when(i == nt - 1)
    def _():
        hf_ref[...] = jnp.dot(perm, h_fin, preferred_element_type=jnp.float32)
        cf_ref[...] = jnp.dot(perm, c_fin, preferred_element_type=jnp.float32)


def _lstm_call(x, lens_col, rest_row, w_ih, w_hh, b_ih2, b_hh2):
    grid = (S // CHUNK,)
    return pl.pallas_call(
        _lstm_body,
        grid=grid,
        in_specs=[
            pl.BlockSpec((B, CHUNK, D), lambda i: (0, i, 0)),
            pl.BlockSpec((B, 1), lambda i: (0, 0)),
            pl.BlockSpec((1, B), lambda i: (0, 0)),
            pl.BlockSpec((G4, D), lambda i: (0, 0)),
            pl.BlockSpec((G4, H), lambda i: (0, 0)),
            pl.BlockSpec((1, G4), lambda i: (0, 0)),
            pl.BlockSpec((1, G4), lambda i: (0, 0)),
        ],
        out_specs=[
            pl.BlockSpec((B, CHUNK, H), lambda i: (0, i, 0)),
            pl.BlockSpec((B, H), lambda i: (0, 0)),
            pl.BlockSpec((B, H), lambda i: (0, 0)),
        ],
        out_shape=[
            jax.ShapeDtypeStruct((B, S, H), jnp.float32),
            jax.ShapeDtypeStruct((B, H), jnp.float32),
            jax.ShapeDtypeStruct((B, H), jnp.float32),
        ],
        scratch_shapes=[
            pltpu.VMEM((B, H), jnp.float32),
            pltpu.VMEM((B, H), jnp.float32),
            pltpu.VMEM((CHUNK, B, G4), jnp.float32),
            pltpu.VMEM((B, CHUNK, H), jnp.float32),
        ],
        compiler_params=pltpu.CompilerParams(
            dimension_semantics=("arbitrary",)),
    )(x, lens_col, rest_row, w_ih, w_hh, b_ih2, b_hh2)


def kernel(inputs, mask, W_ih, W_hh, b_ih, b_hh):
    maskT_i32 = mask.astype(jnp.int32).T
    rest, lengths = _sc_sort(maskT_i32)
    lens_col = lengths.astype(jnp.float32).reshape(B, 1)
    rest_row = rest.reshape(1, B)
    out, hf, cf = _lstm_call(inputs, lens_col, rest_row, W_ih, W_hh,
                             b_ih.reshape(1, G4), b_hh.reshape(1, G4))
    return out, hf[None], cf[None], rest


# speedup vs baseline: 7.8689x; 1.0921x over previous
import functools

import jax
import jax.numpy as jnp
from jax import lax
from jax.experimental import pallas as pl
from jax.experimental.pallas import tpu as pltpu
from jax.experimental.pallas import tpu_sc as plsc

B, S, D, H = 16, 2048, 512, 512
G4 = 4 * H
CHUNK = 64


def _sc_sort(maskT_i32):
    mesh = plsc.VectorSubcoreMesh(core_axis_name="c", subcore_axis_name="s")

    @functools.partial(
        pl.kernel,
        mesh=mesh,
        out_type=(
            jax.ShapeDtypeStruct((B,), jnp.int32),
            jax.ShapeDtypeStruct((B,), jnp.int32),
        ),
        scratch_types=[
            pltpu.VMEM((128, B), jnp.int32),
            pltpu.VMEM((B,), jnp.int32),
            pltpu.VMEM((B,), jnp.int32),
        ],
    )
    def k(mask_hbm, rest_hbm, len_hbm, mask_v, rest_v, len_v):
        wid = lax.axis_index("s") * 2 + lax.axis_index("c")

        @pl.when(wid == 0)
        def _():
            lengths = jnp.zeros((B,), jnp.int32)
            for c in range(S // 128):
                pltpu.sync_copy(mask_hbm.at[pl.ds(c * 128, 128)], mask_v)

                def body(t, acc):
                    return acc + mask_v[t, :]

                lengths = lax.fori_loop(0, 128, body, lengths)
            idx = lax.iota(jnp.int32, B)
            keys = lengths * B + (B - 1) - idx
            rank = jnp.zeros((B,), jnp.int32)
            one = jnp.ones((B,), jnp.int32)
            zero = jnp.zeros((B,), jnp.int32)
            for j in range(B):
                kj = keys[j]
                rank = rank + jnp.where(kj > keys, one, zero)
            rest_v[...] = rank
            len_v[...] = lengths
            pltpu.sync_copy(rest_v, rest_hbm)
            pltpu.sync_copy(len_v, len_hbm)

    return k(maskT_i32)


def _lstm_body(x_ref, len_ref, rest_ref, wih_ref, whh_ref, bih_ref, bhh_ref,
               out_ref, hf_ref, cf_ref, h_s, c_s, a_s, res_s):
    i = pl.program_id(0)
    nt = pl.num_programs(0)

    @pl.when(i == 0)
    def _():
        h_s[...] = jnp.zeros((B, H), jnp.float32)
        c_s[...] = jnp.zeros((B, H), jnp.float32)

    xt = jnp.swapaxes(x_ref[...], 0, 1)
    x2 = xt.reshape(CHUNK * B, D)
    bias = bih_ref[...] + bhh_ref[...]
    a = lax.dot_general(x2, wih_ref[...], (((1,), (1,)), ((), ())),
                        preferred_element_type=jnp.float32) + bias
    a_s[...] = a.reshape(CHUNK, B, G4)

    lens = len_ref[...]
    whh = whh_ref[...]
    t0 = (i * CHUNK).astype(jnp.float32)

    def step(t, carry):
        h, c = carry
        gates = a_s[t] + lax.dot_general(
            h, whh, (((1,), (1,)), ((), ())), preferred_element_type=jnp.float32)
        ig = jax.nn.sigmoid(gates[:, 0:H])
        fg = jax.nn.sigmoid(gates[:, H:2 * H])
        gg = jnp.tanh(gates[:, 2 * H:3 * H])
        og = jax.nn.sigmoid(gates[:, 3 * H:4 * H])
        c_new = fg * c + ig * gg
        h_new = og * jnp.tanh(c_new)
        valid = lens > (t0 + t.astype(jnp.float32))
        res_s[:, t, :] = jnp.where(valid, h_new, 0.0)
        return (jnp.where(valid, h_new, h), jnp.where(valid, c_new, c))

    h_fin, c_fin = lax.fori_loop(0, CHUNK, step, (h_s[...], c_s[...]),
                                 unroll=4)
    h_s[...] = h_fin
    c_s[...] = c_fin

    rows = lax.broadcasted_iota(jnp.int32, (B, B), 0)
    perm = (rows == rest_ref[...]).astype(jnp.float32)
    res = res_s[...].reshape(B, CHUNK * H)
    out_ref[...] = jnp.dot(perm, res,
                           preferred_element_type=jnp.float32).reshape(B, CHUNK, H)

    @pl.when(i == nt - 1)
    def _():
        hf_ref[...] = jnp.dot(perm, h_fin, preferred_element_type=jnp.float32)
        cf_ref[...] = jnp.dot(perm, c_fin, preferred_element_type=jnp.float32)


def _lstm_call(x, lens_col, rest_row, w_ih, w_hh, b_ih2, b_hh2):
    grid = (S // CHUNK,)
    return pl.pallas_call(
        _lstm_body,
        grid=grid,
        in_specs=[
            pl.BlockSpec((B, CHUNK, D), lambda i: (0, i, 0)),
            pl.BlockSpec((B, 1), lambda i: (0, 0)),
            pl.BlockSpec((1, B), lambda i: (0, 0)),
            pl.BlockSpec((G4, D), lambda i: (0, 0)),
            pl.BlockSpec((G4, H), lambda i: (0, 0)),
            pl.BlockSpec((1, G4), lambda i: (0, 0)),
            pl.BlockSpec((1, G4), lambda i: (0, 0)),
        ],
        out_specs=[
            pl.BlockSpec((B, CHUNK, H), lambda i: (0, i, 0)),
            pl.BlockSpec((B, H), lambda i: (0, 0)),
            pl.BlockSpec((B, H), lambda i: (0, 0)),
        ],
        out_shape=[
            jax.ShapeDtypeStruct((B, S, H), jnp.float32),
            jax.ShapeDtypeStruct((B, H), jnp.float32),
            jax.ShapeDtypeStruct((B, H), jnp.float32),
        ],
        scratch_shapes=[
            pltpu.VMEM((B, H), jnp.float32),
            pltpu.VMEM((B, H), jnp.float32),
            pltpu.VMEM((CHUNK, B, G4), jnp.float32),
            pltpu.VMEM((B, CHUNK, H), jnp.float32),
        ],
        compiler_params=pltpu.CompilerParams(
            dimension_semantics=("arbitrary",)),
    )(x, lens_col, rest_row, w_ih, w_hh, b_ih2, b_hh2)


def kernel(inputs, mask, W_ih, W_hh, b_ih, b_hh):
    maskT_i32 = mask.astype(jnp.int32).T
    rest, lengths = _sc_sort(maskT_i32)
    lens_col = lengths.astype(jnp.float32).reshape(B, 1)
    rest_row = rest.reshape(1, B)
    out, hf, cf = _lstm_call(inputs, lens_col, rest_row, W_ih, W_hh,
                             b_ih.reshape(1, G4), b_hh.reshape(1, G4))
    return out, hf[None], cf[None], rest
